# Initial kernel scaffold; baseline (speedup 1.0000x reference)
import jax
import jax.numpy as jnp
from jax.experimental import pallas as pl


def kernel(x, table, W, b):
    raise NotImplementedError("write your pallas kernel here")


# speedup vs baseline: 6.4160x; 6.4160x over previous
import functools

import jax
import jax.numpy as jnp
from jax import lax
from jax.experimental import pallas as pl
from jax.experimental.pallas import tpu as pltpu
from jax.experimental.pallas import tpu_sc as plsc

VOCAB = 100000
D = 128
B_ROWS = 4096 * 200


_PROJ_BLK = 2000


def _proj_body(t_ref, w_ref, b_ref, o_ref):
    o_ref[...] = lax.dot_general(
        t_ref[...], w_ref[...],
        (((1,), (1,)), ((), ())),
        preferred_element_type=jnp.float32,
    ) + b_ref[...]


def _project_table(table, W, b):
    grid = (VOCAB // _PROJ_BLK,)
    return pl.pallas_call(
        _proj_body,
        grid=grid,
        in_specs=[
            pl.BlockSpec((_PROJ_BLK, D), lambda i: (i, 0)),
            pl.BlockSpec((D, D), lambda i: (0, 0)),
            pl.BlockSpec((1, D), lambda i: (0, 0)),
        ],
        out_specs=pl.BlockSpec((_PROJ_BLK, D), lambda i: (i, 0)),
        out_shape=jax.ShapeDtypeStruct((VOCAB, D), jnp.float32),
    )(table, W, b.reshape(1, D))


_NW = 32
_CHUNK = 128
_PER_W = B_ROWS // _NW
_NCHUNK = _PER_W // _CHUNK


def _gather_body(ptab_hbm, xw_hbm, out_hbm, idx_v, rows_v, sem):
    wid = lax.axis_index("s") * 2 + lax.axis_index("c")
    pltpu.sync_copy(xw_hbm.at[pl.ds(wid * _NCHUNK, _NCHUNK)], idx_v)
    base_row = wid * _PER_W

    def body(j, carry):
        pltpu.async_copy(ptab_hbm.at[idx_v.at[j]], rows_v, sem).wait()
        pltpu.sync_copy(rows_v, out_hbm.at[pl.ds(base_row + j * _CHUNK, _CHUNK)])
        return carry

    lax.fori_loop(0, _NCHUNK, body, 0, unroll=False)


def _sc_gather(ptab, x_flat2d):
    mesh = plsc.VectorSubcoreMesh(core_axis_name="c", subcore_axis_name="s")
    kern = functools.partial(
        pl.kernel,
        mesh=mesh,
        out_type=jax.ShapeDtypeStruct((B_ROWS, D), jnp.float32),
        scratch_types=[
            pltpu.VMEM((_NCHUNK, _CHUNK), jnp.int32),
            pltpu.VMEM((_CHUNK, D), jnp.float32),
            pltpu.SemaphoreType.DMA,
        ],
    )(_gather_body)
    return kern(ptab, x_flat2d)


def kernel(x, table, W, b):
    ptab = _project_table(table, W, b)
    x_flat2d = x.reshape(B_ROWS // _CHUNK, _CHUNK).astype(jnp.int32)
    out = _sc_gather(ptab, x_flat2d)
    return out.reshape(x.shape[0], x.shape[1], D)


# speedup vs baseline: 8.9252x; 1.3911x over previous
import functools

import jax
import jax.numpy as jnp
from jax import lax
from jax.experimental import pallas as pl
from jax.experimental.pallas import tpu as pltpu
from jax.experimental.pallas import tpu_sc as plsc

VOCAB = 100000
D = 128
B_ROWS = 4096 * 200


_PROJ_BLK = 2000


def _proj_body(t_ref, w_ref, b_ref, o_ref):
    o_ref[...] = lax.dot_general(
        t_ref[...], w_ref[...],
        (((1,), (1,)), ((), ())),
        preferred_element_type=jnp.float32,
    ) + b_ref[...]


def _project_table(table, W, b):
    grid = (VOCAB // _PROJ_BLK,)
    return pl.pallas_call(
        _proj_body,
        grid=grid,
        in_specs=[
            pl.BlockSpec((_PROJ_BLK, D), lambda i: (i, 0)),
            pl.BlockSpec((D, D), lambda i: (0, 0)),
            pl.BlockSpec((1, D), lambda i: (0, 0)),
        ],
        out_specs=pl.BlockSpec((_PROJ_BLK, D), lambda i: (i, 0)),
        out_shape=jax.ShapeDtypeStruct((VOCAB, D), jnp.float32),
    )(table, W, b.reshape(1, D))


_NW = 32
_CHUNK = 128
_PER_W = B_ROWS // _NW
_NCHUNK = _PER_W // _CHUNK


def _gather_body(ptab_hbm, xw_hbm, out_hbm, idx_v, rows0, rows1, g0, g1, s0, s1):
    wid = lax.axis_index("s") * 2 + lax.axis_index("c")
    pltpu.sync_copy(xw_hbm.at[pl.ds(wid * _NCHUNK, _NCHUNK)], idx_v)
    base_row = wid * _PER_W
    rows = (rows0, rows1)
    gsem = (g0, g1)
    ssem = (s0, s1)

    def gather_start(j, b):
        pltpu.async_copy(ptab_hbm.at[idx_v.at[j]], rows[b], gsem[b])

    def gather_drain(b):
        pltpu.make_async_copy(ptab_hbm.at[pl.ds(0, _CHUNK)], rows[b], gsem[b]).wait()

    def store_start(j, b):
        pltpu.async_copy(rows[b], out_hbm.at[pl.ds(base_row + j * _CHUNK, _CHUNK)],
                         ssem[b])

    def store_drain(b):
        pltpu.make_async_copy(rows[b], out_hbm.at[pl.ds(0, _CHUNK)], ssem[b]).wait()

    gather_start(0, 0)

    def body(o, carry):
        for blk in range(2):
            j = o * 2 + blk
            s = blk
            t = 1 - blk

            @pl.when(jnp.logical_and(j + 1 < _NCHUNK, j >= 1))
            def _():
                store_drain(t)

            @pl.when(j + 1 < _NCHUNK)
            def _():
                gather_start(j + 1, t)

            gather_drain(s)
            store_start(j, s)
        return carry

    lax.fori_loop(0, _NCHUNK // 2, body, 0, unroll=False)
    store_drain(0)
    store_drain(1)


def _sc_gather(ptab, x_flat2d):
    mesh = plsc.VectorSubcoreMesh(core_axis_name="c", subcore_axis_name="s")
    kern = functools.partial(
        pl.kernel,
        mesh=mesh,
        out_type=jax.ShapeDtypeStruct((B_ROWS, D), jnp.float32),
        scratch_types=[
            pltpu.VMEM((_NCHUNK, _CHUNK), jnp.int32),
            pltpu.VMEM((_CHUNK, D), jnp.float32),
            pltpu.VMEM((_CHUNK, D), jnp.float32),
            pltpu.SemaphoreType.DMA,
            pltpu.SemaphoreType.DMA,
            pltpu.SemaphoreType.DMA,
            pltpu.SemaphoreType.DMA,
        ],
    )(_gather_body)
    return kern(ptab, x_flat2d)


def kernel(x, table, W, b):
    ptab = _project_table(table, W, b)
    x_flat2d = x.reshape(B_ROWS // _CHUNK, _CHUNK).astype(jnp.int32)
    out = _sc_gather(ptab, x_flat2d)
    return out.reshape(x.shape[0], x.shape[1], D)
